# Initial kernel scaffold; baseline (speedup 1.0000x reference)
import jax
import jax.numpy as jnp
from jax.experimental import pallas as pl


def kernel(z_text_segs, z_audio_segs, z_facial_segs, Wq, bq, Wk, bk, emb):
    raise NotImplementedError("write your pallas kernel here")


# speedup vs baseline: 1.4966x; 1.4966x over previous
import functools

import jax
import jax.numpy as jnp
from jax.experimental import pallas as pl
from jax.experimental.pallas import tpu as pltpu

B, N, H = 1024, 16, 512
H2 = H // 2
EDGE_DIM = 16
THR = 0.4
E_PER = 6 * (N - 1) + 6 * N
NODES_PER = 3 * N

BB = 8


def _builder_kernel(zt_ref, za_ref, zf_ref, wq_ref, bq_ref, wk_ref, bk_ref,
                    emb_ref, nf_ref, ea_ref, ei_ref, bv_ref):
    pid = pl.program_id(0)
    zt = zt_ref[...]
    za = za_ref[...]
    zf = zf_ref[...]

    nodes = jnp.concatenate([zt, za, zf], axis=1)
    nf_ref[...] = nodes.reshape(BB * NODES_PER, H)

    q_in = jnp.concatenate([zt, za], axis=1).reshape(BB * 2 * N, H)
    k_in = jnp.concatenate([za, zf], axis=1).reshape(BB * 2 * N, H)
    q = q_in @ wq_ref[...] + bq_ref[...]
    k = k_in @ wk_ref[...] + bk_ref[...]
    q = q / jnp.maximum(jnp.sqrt(jnp.sum(q * q, axis=-1, keepdims=True)), 1e-12)
    k = k / jnp.maximum(jnp.sqrt(jnp.sum(k * k, axis=-1, keepdims=True)), 1e-12)
    q3 = q.reshape(BB, 2 * N, H2)
    k3 = k.reshape(BB, 2 * N, H2)
    qt, qa = q3[:, :N, :], q3[:, N:, :]
    ka, kf = k3[:, :N, :], k3[:, N:, :]
    qsel = jnp.concatenate([qt, qt, qa], axis=1).reshape(BB * 48, H2)
    ksel = jnp.concatenate([ka, kf, kf], axis=1).reshape(BB * 48, H2)
    disc_col = 1.0 - jax.nn.sigmoid(
        jnp.sum(qsel * ksel, axis=-1, keepdims=True))
    d16 = jnp.broadcast_to(disc_col, (BB * 48, EDGE_DIM))

    ROWS = BB * E_PER
    rr = jax.lax.broadcasted_iota(jnp.int32, (ROWS, BB * 48), 0)
    vv = jax.lax.broadcasted_iota(jnp.int32, (ROWS, BB * 48), 1)
    cr = rr % E_PER
    oh = ((cr >= 90) & (vv == (rr // E_PER) * 48 + (cr - 90) // 2)
          ).astype(jnp.float32)
    disc_r = jax.lax.dot(oh, d16)
    flag_r = disc_r > THR

    row_i = jax.lax.broadcasted_iota(jnp.int32, (ROWS, EDGE_DIM), 0)
    col = jax.lax.broadcasted_iota(jnp.int32, (ROWS, EDGE_DIM), 1)
    c = row_i % E_PER
    zero8 = jnp.zeros((1, 8), jnp.float32)
    e0 = jnp.concatenate([emb_ref[0:1, :], zero8], axis=1)
    e1 = jnp.concatenate([emb_ref[1:2, :], zero8], axis=1)
    e2 = jnp.concatenate([emb_ref[2:3, :], zero8], axis=1)
    e3 = jnp.concatenate([emb_ref[3:4, :], zero8], axis=1)
    e4 = jnp.concatenate([emb_ref[4:5, :], zero8], axis=1)
    et = c // 30
    et_f = et.astype(jnp.float32)
    embpart = jnp.where(et == 0, e0, jnp.where(et == 1, e1, e2))
    tpart = (jnp.where(col < 8, embpart, 0.0)
             + jnp.where(col == 9, 1.0 / N, 0.0)
             + jnp.where(col == 10, 1.0, 0.0)
             + jnp.where(col == 11, et_f / 4.0, 0.0))
    base3 = jnp.where(col < 8, e3, jnp.where(col == 11, 3.0 / 4.0, 0.0))
    base4 = jnp.where(col < 8, e4, jnp.where(col == 11, 4.0 / 4.0, 0.0))
    cpart = jnp.where(col == 8, disc_r, jnp.where(flag_r, base4, base3))
    ea_ref[...] = jnp.where(c < 90, tpart, cpart)

    e_loc = jax.lax.broadcasted_iota(jnp.int32, (2, BB * E_PER), 1)
    r = jax.lax.broadcasted_iota(jnp.int32, (2, BB * E_PER), 0)
    b_glob = pid * BB + e_loc // E_PER
    c = e_loc % E_PER
    p = c % 2
    g = c // 30
    i = (c % 30) // 2
    t_val = g * N + i + jnp.where(r == 0, p, 1 - p)
    cc = c - 90
    m = cc // 32
    j = (cc % 32) // 2
    ao = jnp.where(m == 2, N, 0)
    bo = jnp.where(m == 0, N, 2 * N)
    c_val = j + jnp.where((p + r) % 2 == 0, ao, bo)
    ei_ref[...] = (jnp.where(c < 90, t_val, c_val) + NODES_PER * b_glob)[None]

    br = jax.lax.broadcasted_iota(jnp.int32, (BB, NODES_PER), 0)
    bv_ref[...] = pid * BB + br


@functools.partial(jax.jit, static_argnames=())
def kernel(z_text_segs, z_audio_segs, z_facial_segs, Wq, bq, Wk, bk, emb):
    grid = (B // BB,)
    nf, ea, ei, bv = pl.pallas_call(
        _builder_kernel,
        grid=grid,
        in_specs=[
            pl.BlockSpec((BB, N, H), lambda i: (i, 0, 0)),
            pl.BlockSpec((BB, N, H), lambda i: (i, 0, 0)),
            pl.BlockSpec((BB, N, H), lambda i: (i, 0, 0)),
            pl.BlockSpec((H, H2), lambda i: (0, 0)),
            pl.BlockSpec((1, H2), lambda i: (0, 0)),
            pl.BlockSpec((H, H2), lambda i: (0, 0)),
            pl.BlockSpec((1, H2), lambda i: (0, 0)),
            pl.BlockSpec((5, 8), lambda i: (0, 0)),
        ],
        out_specs=[
            pl.BlockSpec((BB * NODES_PER, H), lambda i: (i, 0)),
            pl.BlockSpec((BB * E_PER, EDGE_DIM), lambda i: (i, 0)),
            pl.BlockSpec((1, 2, BB * E_PER), lambda i: (i, 0, 0)),
            pl.BlockSpec((BB, NODES_PER), lambda i: (i, 0)),
        ],
        out_shape=[
            jax.ShapeDtypeStruct((B * NODES_PER, H), jnp.float32),
            jax.ShapeDtypeStruct((B * E_PER, EDGE_DIM), jnp.float32),
            jax.ShapeDtypeStruct((B // BB, 2, BB * E_PER), jnp.int32),
            jax.ShapeDtypeStruct((B, NODES_PER), jnp.int32),
        ],
        compiler_params=pltpu.CompilerParams(
            dimension_semantics=("arbitrary",),
        ),
    )(z_text_segs, z_audio_segs, z_facial_segs, Wq, bq.reshape(1, H2),
      Wk, bk.reshape(1, H2), emb)
    return nf, ei.transpose(1, 0, 2).reshape(2, -1), ea, bv.reshape(-1)


# speedup vs baseline: 2.2342x; 1.4928x over previous
import jax
import jax.numpy as jnp
from jax.experimental import pallas as pl
from jax.experimental.pallas import tpu as pltpu

B, N, H = 1024, 16, 512
H2 = H // 2
EDGE_DIM = 16
THR = 0.4
E_PER = 6 * (N - 1) + 6 * N
W_PER = E_PER // 2
NODES_PER = 3 * N

BB = 8
E_TOT = B * E_PER


def _main_kernel(zt_ref, za_ref, zf_ref, wq_ref, bq_ref, wk_ref, bk_ref,
                 emb_ref, nf_ref, ea_ref):
    zt = zt_ref[...]
    za = za_ref[...]
    zf = zf_ref[...]

    nodes = jnp.concatenate([zt, za, zf], axis=1)
    nf_ref[...] = nodes.reshape(BB * NODES_PER, H)

    q_in = jnp.concatenate([zt, za], axis=1).reshape(BB * 2 * N, H)
    k_in = jnp.concatenate([za, zf], axis=1).reshape(BB * 2 * N, H)
    q = q_in @ wq_ref[...] + bq_ref[...]
    k = k_in @ wk_ref[...] + bk_ref[...]
    q3 = q.reshape(BB, 2 * N, H2)
    k3 = k.reshape(BB, 2 * N, H2)
    qt, qa = q3[:, :N, :], q3[:, N:, :]
    ka, kf = k3[:, :N, :], k3[:, N:, :]
    qsel = jnp.concatenate([qt, qt, qa], axis=1).reshape(BB * 48, H2)
    ksel = jnp.concatenate([ka, kf, kf], axis=1).reshape(BB * 48, H2)
    ones32 = jnp.ones((H2, 32), jnp.float32)
    nq = (qsel * qsel) @ ones32
    nk = (ksel * ksel) @ ones32
    dots = (qsel * ksel) @ ones32
    cos = (dots * jax.lax.rsqrt(jnp.maximum(nq, 1e-24))
           * jax.lax.rsqrt(jnp.maximum(nk, 1e-24)))
    disc = 1.0 - jax.nn.sigmoid(cos)

    col = jax.lax.broadcasted_iota(jnp.int32, (BB * 48, 32), 1)
    a_col = col % EDGE_DIM
    zero8 = jnp.zeros((1, 8), jnp.float32)
    e3 = jnp.concatenate([emb_ref[3:4, :], zero8, emb_ref[3:4, :], zero8],
                         axis=1)
    e4 = jnp.concatenate([emb_ref[4:5, :], zero8, emb_ref[4:5, :], zero8],
                         axis=1)
    base3 = jnp.where(a_col < 8, e3, jnp.where(a_col == 11, 3.0 / 4.0, 0.0))
    base4 = jnp.where(a_col < 8, e4, jnp.where(a_col == 11, 4.0 / 4.0, 0.0))
    cross = jnp.where(a_col == 8, disc, jnp.where(disc > THR, base4, base3))

    tw = jax.lax.broadcasted_iota(jnp.int32, (45, 32), 0)
    tcol = jax.lax.broadcasted_iota(jnp.int32, (45, 32), 1)
    tc = 2 * tw + (tcol >= EDGE_DIM).astype(jnp.int32)
    ta = tcol % EDGE_DIM
    et = tc // 30
    e0 = jnp.concatenate([emb_ref[0:1, :], zero8, emb_ref[0:1, :], zero8],
                         axis=1)
    e1 = jnp.concatenate([emb_ref[1:2, :], zero8, emb_ref[1:2, :], zero8],
                         axis=1)
    e2 = jnp.concatenate([emb_ref[2:3, :], zero8, emb_ref[2:3, :], zero8],
                         axis=1)
    embpart = jnp.where(et == 0, e0, jnp.where(et == 1, e1, e2))
    temporal = (jnp.where(ta < 8, embpart, 0.0)
                + jnp.where(ta == 9, 1.0 / N, 0.0)
                + jnp.where(ta == 10, 1.0, 0.0)
                + jnp.where(ta == 11, et.astype(jnp.float32) / 4.0, 0.0))

    for s in range(BB):
        ea_ref[pl.ds(s * W_PER, 45), :] = temporal
        ea_ref[pl.ds(s * W_PER + 45, 48), :] = cross[s * 48:(s + 1) * 48, :]


def _index_kernel(ei_ref, bv_ref):
    r = jax.lax.broadcasted_iota(jnp.int32, (2, E_TOT // 128, 128), 0)
    e = (jax.lax.broadcasted_iota(jnp.int32, (2, E_TOT // 128, 128), 1) * 128
         + jax.lax.broadcasted_iota(jnp.int32, (2, E_TOT // 128, 128), 2))
    b = e // E_PER
    c = e % E_PER
    p = c % 2
    g = c // 30
    i = (c % 30) // 2
    t_val = g * N + i + jnp.where(r == 0, p, 1 - p)
    cc = c - 90
    m = cc // 32
    j = (cc % 32) // 2
    ao = jnp.where(m == 2, N, 0)
    bo = jnp.where(m == 0, N, 2 * N)
    c_val = j + jnp.where((p + r) % 2 == 0, ao, bo)
    ei_ref[...] = jnp.where(c < 90, t_val, c_val) + NODES_PER * b
    v = (jax.lax.broadcasted_iota(jnp.int32, (B * NODES_PER // 128, 128), 0)
         * 128
         + jax.lax.broadcasted_iota(jnp.int32, (B * NODES_PER // 128, 128), 1))
    bv_ref[...] = v // NODES_PER


def kernel(z_text_segs, z_audio_segs, z_facial_segs, Wq, bq, Wk, bk, emb):
    nf, ea = pl.pallas_call(
        _main_kernel,
        grid=(B // BB,),
        in_specs=[
            pl.BlockSpec((BB, N, H), lambda i: (i, 0, 0)),
            pl.BlockSpec((BB, N, H), lambda i: (i, 0, 0)),
            pl.BlockSpec((BB, N, H), lambda i: (i, 0, 0)),
            pl.BlockSpec((H, H2), lambda i: (0, 0)),
            pl.BlockSpec((1, H2), lambda i: (0, 0)),
            pl.BlockSpec((H, H2), lambda i: (0, 0)),
            pl.BlockSpec((1, H2), lambda i: (0, 0)),
            pl.BlockSpec((5, 8), lambda i: (0, 0)),
        ],
        out_specs=[
            pl.BlockSpec((BB * NODES_PER, H), lambda i: (i, 0)),
            pl.BlockSpec((BB * W_PER, 32), lambda i: (i, 0)),
        ],
        out_shape=[
            jax.ShapeDtypeStruct((B * NODES_PER, H), jnp.float32),
            jax.ShapeDtypeStruct((B * W_PER, 32), jnp.float32),
        ],
        compiler_params=pltpu.CompilerParams(
            dimension_semantics=("arbitrary",),
        ),
    )(z_text_segs, z_audio_segs, z_facial_segs, Wq, bq.reshape(1, H2),
      Wk, bk.reshape(1, H2), emb)
    ei, bv = pl.pallas_call(
        _index_kernel,
        out_shape=[
            jax.ShapeDtypeStruct((2, E_TOT // 128, 128), jnp.int32),
            jax.ShapeDtypeStruct((B * NODES_PER // 128, 128), jnp.int32),
        ],
    )()
    return (nf, ei.reshape(2, E_TOT), ea.reshape(E_TOT, EDGE_DIM),
            bv.reshape(B * NODES_PER))


# speedup vs baseline: 2.7179x; 1.2165x over previous
import jax
import jax.numpy as jnp
from jax.experimental import pallas as pl
from jax.experimental.pallas import tpu as pltpu

B, N, H = 1024, 16, 512
H2 = H // 2
EDGE_DIM = 16
THR = 0.4
E_PER = 6 * (N - 1) + 6 * N
W_PER = E_PER // 2
NODES_PER = 3 * N

BB = 16
E_TOT = B * E_PER


def _main_kernel(zt_ref, za_ref, zf_ref, wq_ref, bq_ref, wk_ref, bk_ref,
                 emb_ref, nf_ref, ea_ref):
    zt = zt_ref[...]
    za = za_ref[...]
    zf = zf_ref[...]

    nodes = jnp.concatenate([zt, za, zf], axis=1)
    nf_ref[...] = nodes.reshape(BB * NODES_PER, H)

    q_in = jnp.concatenate([zt, za], axis=1).reshape(BB * 2 * N, H)
    k_in = jnp.concatenate([za, zf], axis=1).reshape(BB * 2 * N, H)
    q = q_in @ wq_ref[...] + bq_ref[...]
    k = k_in @ wk_ref[...] + bk_ref[...]
    q3 = q.reshape(BB, 2 * N, H2)
    k3 = k.reshape(BB, 2 * N, H2)
    qt, qa = q3[:, :N, :], q3[:, N:, :]
    ka, kf = k3[:, :N, :], k3[:, N:, :]
    qsel = jnp.concatenate([qt, qt, qa], axis=1).reshape(BB * 48, H2)
    ksel = jnp.concatenate([ka, kf, kf], axis=1).reshape(BB * 48, H2)
    ones32 = jnp.ones((H2, 32), jnp.float32)
    nq = (qsel * qsel) @ ones32
    nk = (ksel * ksel) @ ones32
    dots = (qsel * ksel) @ ones32
    cos = (dots * jax.lax.rsqrt(jnp.maximum(nq, 1e-24))
           * jax.lax.rsqrt(jnp.maximum(nk, 1e-24)))
    disc = 1.0 - jax.nn.sigmoid(cos)

    col = jax.lax.broadcasted_iota(jnp.int32, (BB * 48, 32), 1)
    a_col = col % EDGE_DIM
    zero8 = jnp.zeros((1, 8), jnp.float32)
    e3 = jnp.concatenate([emb_ref[3:4, :], zero8, emb_ref[3:4, :], zero8],
                         axis=1)
    e4 = jnp.concatenate([emb_ref[4:5, :], zero8, emb_ref[4:5, :], zero8],
                         axis=1)
    base3 = jnp.where(a_col < 8, e3, jnp.where(a_col == 11, 3.0 / 4.0, 0.0))
    base4 = jnp.where(a_col < 8, e4, jnp.where(a_col == 11, 4.0 / 4.0, 0.0))
    cross = jnp.where(a_col == 8, disc, jnp.where(disc > THR, base4, base3))

    tw = jax.lax.broadcasted_iota(jnp.int32, (45, 32), 0)
    tcol = jax.lax.broadcasted_iota(jnp.int32, (45, 32), 1)
    tc = 2 * tw + (tcol >= EDGE_DIM).astype(jnp.int32)
    ta = tcol % EDGE_DIM
    et = tc // 30
    e0 = jnp.concatenate([emb_ref[0:1, :], zero8, emb_ref[0:1, :], zero8],
                         axis=1)
    e1 = jnp.concatenate([emb_ref[1:2, :], zero8, emb_ref[1:2, :], zero8],
                         axis=1)
    e2 = jnp.concatenate([emb_ref[2:3, :], zero8, emb_ref[2:3, :], zero8],
                         axis=1)
    embpart = jnp.where(et == 0, e0, jnp.where(et == 1, e1, e2))
    temporal = (jnp.where(ta < 8, embpart, 0.0)
                + jnp.where(ta == 9, 1.0 / N, 0.0)
                + jnp.where(ta == 10, 1.0, 0.0)
                + jnp.where(ta == 11, et.astype(jnp.float32) / 4.0, 0.0))

    for s in range(BB):
        ea_ref[pl.ds(s * W_PER, 45), :] = temporal
        ea_ref[pl.ds(s * W_PER + 45, 48), :] = cross[s * 48:(s + 1) * 48, :]


def _index_kernel(ei_ref, bv_ref):
    r = jax.lax.broadcasted_iota(jnp.int32, (2, E_TOT // 128, 128), 0)
    e = (jax.lax.broadcasted_iota(jnp.int32, (2, E_TOT // 128, 128), 1) * 128
         + jax.lax.broadcasted_iota(jnp.int32, (2, E_TOT // 128, 128), 2))
    b = e // E_PER
    c = e % E_PER
    p = c % 2
    g = c // 30
    i = (c % 30) // 2
    t_val = g * N + i + jnp.where(r == 0, p, 1 - p)
    cc = c - 90
    m = cc // 32
    j = (cc % 32) // 2
    ao = jnp.where(m == 2, N, 0)
    bo = jnp.where(m == 0, N, 2 * N)
    c_val = j + jnp.where((p + r) % 2 == 0, ao, bo)
    ei_ref[...] = jnp.where(c < 90, t_val, c_val) + NODES_PER * b
    v = (jax.lax.broadcasted_iota(jnp.int32, (B * NODES_PER // 128, 128), 0)
         * 128
         + jax.lax.broadcasted_iota(jnp.int32, (B * NODES_PER // 128, 128), 1))
    bv_ref[...] = v // NODES_PER


def kernel(z_text_segs, z_audio_segs, z_facial_segs, Wq, bq, Wk, bk, emb):
    nf, ea = pl.pallas_call(
        _main_kernel,
        grid=(B // BB,),
        in_specs=[
            pl.BlockSpec((BB, N, H), lambda i: (i, 0, 0)),
            pl.BlockSpec((BB, N, H), lambda i: (i, 0, 0)),
            pl.BlockSpec((BB, N, H), lambda i: (i, 0, 0)),
            pl.BlockSpec((H, H2), lambda i: (0, 0)),
            pl.BlockSpec((1, H2), lambda i: (0, 0)),
            pl.BlockSpec((H, H2), lambda i: (0, 0)),
            pl.BlockSpec((1, H2), lambda i: (0, 0)),
            pl.BlockSpec((5, 8), lambda i: (0, 0)),
        ],
        out_specs=[
            pl.BlockSpec((BB * NODES_PER, H), lambda i: (i, 0)),
            pl.BlockSpec((BB * W_PER, 32), lambda i: (i, 0)),
        ],
        out_shape=[
            jax.ShapeDtypeStruct((B * NODES_PER, H), jnp.float32),
            jax.ShapeDtypeStruct((B * W_PER, 32), jnp.float32),
        ],
        compiler_params=pltpu.CompilerParams(
            dimension_semantics=("arbitrary",),
        ),
    )(z_text_segs, z_audio_segs, z_facial_segs, Wq, bq.reshape(1, H2),
      Wk, bk.reshape(1, H2), emb)
    ei, bv = pl.pallas_call(
        _index_kernel,
        out_shape=[
            jax.ShapeDtypeStruct((2, E_TOT // 128, 128), jnp.int32),
            jax.ShapeDtypeStruct((B * NODES_PER // 128, 128), jnp.int32),
        ],
    )()
    return (nf, ei.reshape(2, E_TOT), ea.reshape(E_TOT, EDGE_DIM),
            bv.reshape(B * NODES_PER))


# speedup vs baseline: 2.9677x; 1.0919x over previous
import jax
import jax.numpy as jnp
from jax.experimental import pallas as pl
from jax.experimental.pallas import tpu as pltpu

B, N, H = 1024, 16, 512
H2 = H // 2
EDGE_DIM = 16
THR = 0.4
E_PER = 6 * (N - 1) + 6 * N
W_PER = E_PER // 2
NODES_PER = 3 * N

BB = 32
E_TOT = B * E_PER


def _main_kernel(zt_ref, za_ref, zf_ref, wq_ref, bq_ref, wk_ref, bk_ref,
                 emb_ref, nf_ref, ea_ref):
    zt = zt_ref[...]
    za = za_ref[...]
    zf = zf_ref[...]

    nodes = jnp.concatenate([zt, za, zf], axis=1)
    nf_ref[...] = nodes.reshape(BB * NODES_PER, H)

    q_in = jnp.concatenate([zt, za], axis=1).reshape(BB * 2 * N, H)
    k_in = jnp.concatenate([za, zf], axis=1).reshape(BB * 2 * N, H)
    q = q_in @ wq_ref[...] + bq_ref[...]
    k = k_in @ wk_ref[...] + bk_ref[...]
    q3 = q.reshape(BB, 2 * N, H2)
    k3 = k.reshape(BB, 2 * N, H2)
    qt, qa = q3[:, :N, :], q3[:, N:, :]
    ka, kf = k3[:, :N, :], k3[:, N:, :]
    qsel = jnp.concatenate([qt, qt, qa], axis=1).reshape(BB * 48, H2)
    ksel = jnp.concatenate([ka, kf, kf], axis=1).reshape(BB * 48, H2)
    ones32 = jnp.ones((H2, 32), jnp.float32)
    nq = (qsel * qsel) @ ones32
    nk = (ksel * ksel) @ ones32
    dots = (qsel * ksel) @ ones32
    cos = (dots * jax.lax.rsqrt(jnp.maximum(nq, 1e-24))
           * jax.lax.rsqrt(jnp.maximum(nk, 1e-24)))
    disc = 1.0 - jax.nn.sigmoid(cos)

    col = jax.lax.broadcasted_iota(jnp.int32, (BB * 48, 32), 1)
    a_col = col % EDGE_DIM
    zero8 = jnp.zeros((1, 8), jnp.float32)
    e3 = jnp.concatenate([emb_ref[3:4, :], zero8, emb_ref[3:4, :], zero8],
                         axis=1)
    e4 = jnp.concatenate([emb_ref[4:5, :], zero8, emb_ref[4:5, :], zero8],
                         axis=1)
    base3 = jnp.where(a_col < 8, e3, jnp.where(a_col == 11, 3.0 / 4.0, 0.0))
    base4 = jnp.where(a_col < 8, e4, jnp.where(a_col == 11, 4.0 / 4.0, 0.0))
    cross = jnp.where(a_col == 8, disc, jnp.where(disc > THR, base4, base3))

    tw = jax.lax.broadcasted_iota(jnp.int32, (45, 32), 0)
    tcol = jax.lax.broadcasted_iota(jnp.int32, (45, 32), 1)
    tc = 2 * tw + (tcol >= EDGE_DIM).astype(jnp.int32)
    ta = tcol % EDGE_DIM
    et = tc // 30
    e0 = jnp.concatenate([emb_ref[0:1, :], zero8, emb_ref[0:1, :], zero8],
                         axis=1)
    e1 = jnp.concatenate([emb_ref[1:2, :], zero8, emb_ref[1:2, :], zero8],
                         axis=1)
    e2 = jnp.concatenate([emb_ref[2:3, :], zero8, emb_ref[2:3, :], zero8],
                         axis=1)
    embpart = jnp.where(et == 0, e0, jnp.where(et == 1, e1, e2))
    temporal = (jnp.where(ta < 8, embpart, 0.0)
                + jnp.where(ta == 9, 1.0 / N, 0.0)
                + jnp.where(ta == 10, 1.0, 0.0)
                + jnp.where(ta == 11, et.astype(jnp.float32) / 4.0, 0.0))

    for s in range(BB):
        ea_ref[pl.ds(s * W_PER, 45), :] = temporal
        ea_ref[pl.ds(s * W_PER + 45, 48), :] = cross[s * 48:(s + 1) * 48, :]


def _index_kernel(ei_ref, bv_ref):
    r = jax.lax.broadcasted_iota(jnp.int32, (2, E_TOT // 128, 128), 0)
    e = (jax.lax.broadcasted_iota(jnp.int32, (2, E_TOT // 128, 128), 1) * 128
         + jax.lax.broadcasted_iota(jnp.int32, (2, E_TOT // 128, 128), 2))
    b = e // E_PER
    c = e % E_PER
    p = c % 2
    g = c // 30
    i = (c % 30) // 2
    t_val = g * N + i + jnp.where(r == 0, p, 1 - p)
    cc = c - 90
    m = cc // 32
    j = (cc % 32) // 2
    ao = jnp.where(m == 2, N, 0)
    bo = jnp.where(m == 0, N, 2 * N)
    c_val = j + jnp.where((p + r) % 2 == 0, ao, bo)
    ei_ref[...] = jnp.where(c < 90, t_val, c_val) + NODES_PER * b
    v = (jax.lax.broadcasted_iota(jnp.int32, (B * NODES_PER // 128, 128), 0)
         * 128
         + jax.lax.broadcasted_iota(jnp.int32, (B * NODES_PER // 128, 128), 1))
    bv_ref[...] = v // NODES_PER


def kernel(z_text_segs, z_audio_segs, z_facial_segs, Wq, bq, Wk, bk, emb):
    nf, ea = pl.pallas_call(
        _main_kernel,
        grid=(B // BB,),
        in_specs=[
            pl.BlockSpec((BB, N, H), lambda i: (i, 0, 0)),
            pl.BlockSpec((BB, N, H), lambda i: (i, 0, 0)),
            pl.BlockSpec((BB, N, H), lambda i: (i, 0, 0)),
            pl.BlockSpec((H, H2), lambda i: (0, 0)),
            pl.BlockSpec((1, H2), lambda i: (0, 0)),
            pl.BlockSpec((H, H2), lambda i: (0, 0)),
            pl.BlockSpec((1, H2), lambda i: (0, 0)),
            pl.BlockSpec((5, 8), lambda i: (0, 0)),
        ],
        out_specs=[
            pl.BlockSpec((BB * NODES_PER, H), lambda i: (i, 0)),
            pl.BlockSpec((BB * W_PER, 32), lambda i: (i, 0)),
        ],
        out_shape=[
            jax.ShapeDtypeStruct((B * NODES_PER, H), jnp.float32),
            jax.ShapeDtypeStruct((B * W_PER, 32), jnp.float32),
        ],
        compiler_params=pltpu.CompilerParams(
            dimension_semantics=("arbitrary",),
        ),
    )(z_text_segs, z_audio_segs, z_facial_segs, Wq, bq.reshape(1, H2),
      Wk, bk.reshape(1, H2), emb)
    ei, bv = pl.pallas_call(
        _index_kernel,
        out_shape=[
            jax.ShapeDtypeStruct((2, E_TOT // 128, 128), jnp.int32),
            jax.ShapeDtypeStruct((B * NODES_PER // 128, 128), jnp.int32),
        ],
    )()
    return (nf, ei.reshape(2, E_TOT), ea.reshape(E_TOT, EDGE_DIM),
            bv.reshape(B * NODES_PER))


# speedup vs baseline: 3.0284x; 1.0205x over previous
import jax
import jax.numpy as jnp
from jax.experimental import pallas as pl
from jax.experimental.pallas import tpu as pltpu

B, N, H = 1024, 16, 512
H2 = H // 2
EDGE_DIM = 16
THR = 0.4
E_PER = 6 * (N - 1) + 6 * N
W_PER = E_PER // 2
NODES_PER = 3 * N

BB = 64
E_TOT = B * E_PER


def _main_kernel(zt_ref, za_ref, zf_ref, wq_ref, bq_ref, wk_ref, bk_ref,
                 emb_ref, nf_ref, ea_ref):
    zt = zt_ref[...]
    za = za_ref[...]
    zf = zf_ref[...]

    nodes = jnp.concatenate([zt, za, zf], axis=1)
    nf_ref[...] = nodes.reshape(BB * NODES_PER, H)

    q_in = jnp.concatenate([zt, za], axis=1).reshape(BB * 2 * N, H)
    k_in = jnp.concatenate([za, zf], axis=1).reshape(BB * 2 * N, H)
    q = q_in @ wq_ref[...] + bq_ref[...]
    k = k_in @ wk_ref[...] + bk_ref[...]
    q3 = q.reshape(BB, 2 * N, H2)
    k3 = k.reshape(BB, 2 * N, H2)
    qt, qa = q3[:, :N, :], q3[:, N:, :]
    ka, kf = k3[:, :N, :], k3[:, N:, :]
    qsel = jnp.concatenate([qt, qt, qa], axis=1).reshape(BB * 48, H2)
    ksel = jnp.concatenate([ka, kf, kf], axis=1).reshape(BB * 48, H2)
    ones32 = jnp.ones((H2, 32), jnp.float32)
    nq = (qsel * qsel) @ ones32
    nk = (ksel * ksel) @ ones32
    dots = (qsel * ksel) @ ones32
    cos = (dots * jax.lax.rsqrt(jnp.maximum(nq, 1e-24))
           * jax.lax.rsqrt(jnp.maximum(nk, 1e-24)))
    disc = 1.0 - jax.nn.sigmoid(cos)

    col = jax.lax.broadcasted_iota(jnp.int32, (BB * 48, 32), 1)
    a_col = col % EDGE_DIM
    zero8 = jnp.zeros((1, 8), jnp.float32)
    e3 = jnp.concatenate([emb_ref[3:4, :], zero8, emb_ref[3:4, :], zero8],
                         axis=1)
    e4 = jnp.concatenate([emb_ref[4:5, :], zero8, emb_ref[4:5, :], zero8],
                         axis=1)
    base3 = jnp.where(a_col < 8, e3, jnp.where(a_col == 11, 3.0 / 4.0, 0.0))
    base4 = jnp.where(a_col < 8, e4, jnp.where(a_col == 11, 4.0 / 4.0, 0.0))
    cross = jnp.where(a_col == 8, disc, jnp.where(disc > THR, base4, base3))

    tw = jax.lax.broadcasted_iota(jnp.int32, (45, 32), 0)
    tcol = jax.lax.broadcasted_iota(jnp.int32, (45, 32), 1)
    tc = 2 * tw + (tcol >= EDGE_DIM).astype(jnp.int32)
    ta = tcol % EDGE_DIM
    et = tc // 30
    e0 = jnp.concatenate([emb_ref[0:1, :], zero8, emb_ref[0:1, :], zero8],
                         axis=1)
    e1 = jnp.concatenate([emb_ref[1:2, :], zero8, emb_ref[1:2, :], zero8],
                         axis=1)
    e2 = jnp.concatenate([emb_ref[2:3, :], zero8, emb_ref[2:3, :], zero8],
                         axis=1)
    embpart = jnp.where(et == 0, e0, jnp.where(et == 1, e1, e2))
    temporal = (jnp.where(ta < 8, embpart, 0.0)
                + jnp.where(ta == 9, 1.0 / N, 0.0)
                + jnp.where(ta == 10, 1.0, 0.0)
                + jnp.where(ta == 11, et.astype(jnp.float32) / 4.0, 0.0))

    for s in range(BB):
        ea_ref[pl.ds(s * W_PER, 45), :] = temporal
        ea_ref[pl.ds(s * W_PER + 45, 48), :] = cross[s * 48:(s + 1) * 48, :]


def _index_kernel(ei_ref, bv_ref):
    r = jax.lax.broadcasted_iota(jnp.int32, (2, E_TOT // 128, 128), 0)
    e = (jax.lax.broadcasted_iota(jnp.int32, (2, E_TOT // 128, 128), 1) * 128
         + jax.lax.broadcasted_iota(jnp.int32, (2, E_TOT // 128, 128), 2))
    b = e // E_PER
    c = e % E_PER
    p = c % 2
    g = c // 30
    i = (c % 30) // 2
    t_val = g * N + i + jnp.where(r == 0, p, 1 - p)
    cc = c - 90
    m = cc // 32
    j = (cc % 32) // 2
    ao = jnp.where(m == 2, N, 0)
    bo = jnp.where(m == 0, N, 2 * N)
    c_val = j + jnp.where((p + r) % 2 == 0, ao, bo)
    ei_ref[...] = jnp.where(c < 90, t_val, c_val) + NODES_PER * b
    v = (jax.lax.broadcasted_iota(jnp.int32, (B * NODES_PER // 128, 128), 0)
         * 128
         + jax.lax.broadcasted_iota(jnp.int32, (B * NODES_PER // 128, 128), 1))
    bv_ref[...] = v // NODES_PER


def kernel(z_text_segs, z_audio_segs, z_facial_segs, Wq, bq, Wk, bk, emb):
    nf, ea = pl.pallas_call(
        _main_kernel,
        grid=(B // BB,),
        in_specs=[
            pl.BlockSpec((BB, N, H), lambda i: (i, 0, 0)),
            pl.BlockSpec((BB, N, H), lambda i: (i, 0, 0)),
            pl.BlockSpec((BB, N, H), lambda i: (i, 0, 0)),
            pl.BlockSpec((H, H2), lambda i: (0, 0)),
            pl.BlockSpec((1, H2), lambda i: (0, 0)),
            pl.BlockSpec((H, H2), lambda i: (0, 0)),
            pl.BlockSpec((1, H2), lambda i: (0, 0)),
            pl.BlockSpec((5, 8), lambda i: (0, 0)),
        ],
        out_specs=[
            pl.BlockSpec((BB * NODES_PER, H), lambda i: (i, 0)),
            pl.BlockSpec((BB * W_PER, 32), lambda i: (i, 0)),
        ],
        out_shape=[
            jax.ShapeDtypeStruct((B * NODES_PER, H), jnp.float32),
            jax.ShapeDtypeStruct((B * W_PER, 32), jnp.float32),
        ],
        compiler_params=pltpu.CompilerParams(
            dimension_semantics=("arbitrary",),
        ),
    )(z_text_segs, z_audio_segs, z_facial_segs, Wq, bq.reshape(1, H2),
      Wk, bk.reshape(1, H2), emb)
    ei, bv = pl.pallas_call(
        _index_kernel,
        out_shape=[
            jax.ShapeDtypeStruct((2, E_TOT // 128, 128), jnp.int32),
            jax.ShapeDtypeStruct((B * NODES_PER // 128, 128), jnp.int32),
        ],
    )()
    return (nf, ei.reshape(2, E_TOT), ea.reshape(E_TOT, EDGE_DIM),
            bv.reshape(B * NODES_PER))


# speedup vs baseline: 3.0388x; 1.0034x over previous
import jax
import jax.numpy as jnp
from jax.experimental import pallas as pl
from jax.experimental.pallas import tpu as pltpu

B, N, H = 1024, 16, 512
H2 = H // 2
EDGE_DIM = 16
THR = 0.4
E_PER = 6 * (N - 1) + 6 * N
W_PER = E_PER // 2
NODES_PER = 3 * N

BB = 64
E_TOT = B * E_PER


def _main_kernel(zt_ref, za_ref, zf_ref, wq_ref, bq_ref, wk_ref, bk_ref,
                 emb_ref, nf_ref, ea_ref):
    zt = zt_ref[...]
    za = za_ref[...]
    zf = zf_ref[...]

    nodes = jnp.concatenate([zt, za, zf], axis=1)
    nf_ref[...] = nodes.reshape(BB * NODES_PER, H)

    q_in = jnp.concatenate([zt, za], axis=1).reshape(BB * 2 * N, H)
    k_in = jnp.concatenate([za, zf], axis=1).reshape(BB * 2 * N, H)
    q = jax.lax.dot(q_in.astype(jnp.bfloat16),
                    wq_ref[...].astype(jnp.bfloat16),
                    preferred_element_type=jnp.float32) + bq_ref[...]
    k = jax.lax.dot(k_in.astype(jnp.bfloat16),
                    wk_ref[...].astype(jnp.bfloat16),
                    preferred_element_type=jnp.float32) + bk_ref[...]
    q3 = q.reshape(BB, 2 * N, H2)
    k3 = k.reshape(BB, 2 * N, H2)
    qt, qa = q3[:, :N, :], q3[:, N:, :]
    ka, kf = k3[:, :N, :], k3[:, N:, :]
    qsel = jnp.concatenate([qt, qt, qa], axis=1).reshape(BB * 48, H2)
    ksel = jnp.concatenate([ka, kf, kf], axis=1).reshape(BB * 48, H2)
    ones32 = jnp.ones((H2, 32), jnp.float32)
    nq = (qsel * qsel) @ ones32
    nk = (ksel * ksel) @ ones32
    dots = (qsel * ksel) @ ones32
    cos = (dots * jax.lax.rsqrt(jnp.maximum(nq, 1e-24))
           * jax.lax.rsqrt(jnp.maximum(nk, 1e-24)))
    disc = 1.0 - jax.nn.sigmoid(cos)

    col = jax.lax.broadcasted_iota(jnp.int32, (BB * 48, 32), 1)
    a_col = col % EDGE_DIM
    zero8 = jnp.zeros((1, 8), jnp.float32)
    e3 = jnp.concatenate([emb_ref[3:4, :], zero8, emb_ref[3:4, :], zero8],
                         axis=1)
    e4 = jnp.concatenate([emb_ref[4:5, :], zero8, emb_ref[4:5, :], zero8],
                         axis=1)
    base3 = jnp.where(a_col < 8, e3, jnp.where(a_col == 11, 3.0 / 4.0, 0.0))
    base4 = jnp.where(a_col < 8, e4, jnp.where(a_col == 11, 4.0 / 4.0, 0.0))
    cross = jnp.where(a_col == 8, disc, jnp.where(disc > THR, base4, base3))

    tw = jax.lax.broadcasted_iota(jnp.int32, (45, 32), 0)
    tcol = jax.lax.broadcasted_iota(jnp.int32, (45, 32), 1)
    tc = 2 * tw + (tcol >= EDGE_DIM).astype(jnp.int32)
    ta = tcol % EDGE_DIM
    et = tc // 30
    e0 = jnp.concatenate([emb_ref[0:1, :], zero8, emb_ref[0:1, :], zero8],
                         axis=1)
    e1 = jnp.concatenate([emb_ref[1:2, :], zero8, emb_ref[1:2, :], zero8],
                         axis=1)
    e2 = jnp.concatenate([emb_ref[2:3, :], zero8, emb_ref[2:3, :], zero8],
                         axis=1)
    embpart = jnp.where(et == 0, e0, jnp.where(et == 1, e1, e2))
    temporal = (jnp.where(ta < 8, embpart, 0.0)
                + jnp.where(ta == 9, 1.0 / N, 0.0)
                + jnp.where(ta == 10, 1.0, 0.0)
                + jnp.where(ta == 11, et.astype(jnp.float32) / 4.0, 0.0))

    for s in range(BB):
        ea_ref[pl.ds(s * W_PER, 45), :] = temporal
        ea_ref[pl.ds(s * W_PER + 45, 48), :] = cross[s * 48:(s + 1) * 48, :]


def _index_kernel(ei_ref, bv_ref):
    r = jax.lax.broadcasted_iota(jnp.int32, (2, E_TOT // 128, 128), 0)
    e = (jax.lax.broadcasted_iota(jnp.int32, (2, E_TOT // 128, 128), 1) * 128
         + jax.lax.broadcasted_iota(jnp.int32, (2, E_TOT // 128, 128), 2))
    b = e // E_PER
    c = e % E_PER
    p = c % 2
    g = c // 30
    i = (c % 30) // 2
    t_val = g * N + i + jnp.where(r == 0, p, 1 - p)
    cc = c - 90
    m = cc // 32
    j = (cc % 32) // 2
    ao = jnp.where(m == 2, N, 0)
    bo = jnp.where(m == 0, N, 2 * N)
    c_val = j + jnp.where((p + r) % 2 == 0, ao, bo)
    ei_ref[...] = jnp.where(c < 90, t_val, c_val) + NODES_PER * b
    v = (jax.lax.broadcasted_iota(jnp.int32, (B * NODES_PER // 128, 128), 0)
         * 128
         + jax.lax.broadcasted_iota(jnp.int32, (B * NODES_PER // 128, 128), 1))
    bv_ref[...] = v // NODES_PER


def kernel(z_text_segs, z_audio_segs, z_facial_segs, Wq, bq, Wk, bk, emb):
    nf, ea = pl.pallas_call(
        _main_kernel,
        grid=(B // BB,),
        in_specs=[
            pl.BlockSpec((BB, N, H), lambda i: (i, 0, 0)),
            pl.BlockSpec((BB, N, H), lambda i: (i, 0, 0)),
            pl.BlockSpec((BB, N, H), lambda i: (i, 0, 0)),
            pl.BlockSpec((H, H2), lambda i: (0, 0)),
            pl.BlockSpec((1, H2), lambda i: (0, 0)),
            pl.BlockSpec((H, H2), lambda i: (0, 0)),
            pl.BlockSpec((1, H2), lambda i: (0, 0)),
            pl.BlockSpec((5, 8), lambda i: (0, 0)),
        ],
        out_specs=[
            pl.BlockSpec((BB * NODES_PER, H), lambda i: (i, 0)),
            pl.BlockSpec((BB * W_PER, 32), lambda i: (i, 0)),
        ],
        out_shape=[
            jax.ShapeDtypeStruct((B * NODES_PER, H), jnp.float32),
            jax.ShapeDtypeStruct((B * W_PER, 32), jnp.float32),
        ],
        compiler_params=pltpu.CompilerParams(
            dimension_semantics=("arbitrary",),
        ),
    )(z_text_segs, z_audio_segs, z_facial_segs, Wq, bq.reshape(1, H2),
      Wk, bk.reshape(1, H2), emb)
    ei, bv = pl.pallas_call(
        _index_kernel,
        out_shape=[
            jax.ShapeDtypeStruct((2, E_TOT // 128, 128), jnp.int32),
            jax.ShapeDtypeStruct((B * NODES_PER // 128, 128), jnp.int32),
        ],
    )()
    return (nf, ei.reshape(2, E_TOT), ea.reshape(E_TOT, EDGE_DIM),
            bv.reshape(B * NODES_PER))
